# baseline (speedup 1.0000x reference)
import functools

import jax
import jax.numpy as jnp
from jax import lax
from jax.experimental import pallas as pl
from jax.experimental.pallas import tpu as pltpu
from jax.experimental.pallas import tpu_sc as plsc

NUM_ITEMS = 1000000
LATENT_DIM = 64
BATCH = 16384

NC = 2
NS = 16
L = 16
NW = NC * NS
BPW = BATCH // NW
CHUNK = 128
NCHUNK = BPW // CHUNK


def _body(table_hbm, idx_hbm, w_hbm, b_hbm, out_hbm,
          idx_v, rows_v, out_v, w_v, b_v, tr_v, sem):
    wid = lax.axis_index("s") * NC + lax.axis_index("c")
    base = wid * BPW

    pltpu.sync_copy(w_hbm, w_v)
    pltpu.sync_copy(b_hbm, b_v)
    for c in range(NCHUNK):
        pltpu.sync_copy(idx_hbm.at[pl.ds(base + c * CHUNK, CHUNK)],
                        idx_v.at[c])
    copies = []
    for c in range(NCHUNK):
        copies.append(pltpu.async_copy(
            table_hbm.at[idx_v.at[c]],
            rows_v.at[pl.ds(c * CHUNK, CHUNK)], sem))
    for cp in copies:
        cp.wait()

    w0 = w_v[pl.ds(0, L)]
    w1 = w_v[pl.ds(L, L)]
    w2 = w_v[pl.ds(2 * L, L)]
    w3 = w_v[pl.ds(3 * L, L)]
    bv = b_v[...]
    colbase = lax.iota(jnp.int32, L) * L

    def group(g, carry):
        for i in range(L):
            r = g * L + i
            p = bv + rows_v[r, pl.ds(0, L)] * w0
            p = p + rows_v[r, pl.ds(L, L)] * w1
            p = p + rows_v[r, pl.ds(2 * L, L)] * w2
            p = p + rows_v[r, pl.ds(3 * L, L)] * w3
            tr_v[pl.ds(i * L, L)] = p
        s = plsc.load_gather(tr_v, [colbase])
        for l in range(1, L):
            s = s + plsc.load_gather(tr_v, [colbase + l])
        out_v[pl.ds(g * L, L)] = s
        return carry

    lax.fori_loop(0, BPW // L, group, 0)

    pltpu.sync_copy(out_v, out_hbm.at[pl.ds(base, BPW)])


@jax.jit
def kernel(item_indices, item_emb, ffn_w, ffn_b):
    idx = item_indices.astype(jnp.int32)
    w = ffn_w.reshape(LATENT_DIM).astype(jnp.float32)
    bvec = jnp.pad(ffn_b.astype(jnp.float32), (0, L - 1))

    run = pl.kernel(
        _body,
        out_type=jax.ShapeDtypeStruct((BATCH,), jnp.float32),
        mesh=plsc.VectorSubcoreMesh(core_axis_name="c", subcore_axis_name="s",
                                    num_cores=NC, num_subcores=NS),
        compiler_params=pltpu.CompilerParams(needs_layout_passes=False,
                                             use_tc_tiling_on_sc=False),
        scratch_types=[
            pltpu.VMEM((NCHUNK, CHUNK), jnp.int32),
            pltpu.VMEM((BPW, LATENT_DIM), jnp.float32),
            pltpu.VMEM((BPW,), jnp.float32),
            pltpu.VMEM((LATENT_DIM,), jnp.float32),
            pltpu.VMEM((L,), jnp.float32),
            pltpu.VMEM((L * L,), jnp.float32),
            pltpu.SemaphoreType.DMA,
        ],
    )
    out = run(item_emb, idx, w, bvec)
    return out.reshape(BATCH, 1)
